# Initial kernel scaffold; baseline (speedup 1.0000x reference)
import jax
import jax.numpy as jnp
from jax.experimental import pallas as pl


def kernel(x, log_p):
    raise NotImplementedError("write your pallas kernel here")


# speedup vs baseline: 3588.2635x; 3588.2635x over previous
import jax
import jax.numpy as jnp
import numpy as np
from jax import lax
from jax.experimental import pallas as pl
from jax.experimental.pallas import tpu as pltpu
from jax.experimental.pallas import tpu_sc as plsc

INPUT_DIM = 512
NUM_ELMT = 64
RATIO = 1.2
BOUND = 50.0
N = 65536

L = 16
NC = 2
NS = 16
NW = NC * NS
COLS = INPUT_DIM // NW
R = 2048
NCHUNK = N // R
YR_ROWS = 128


def _mesh_tables():
    one_step = BOUND * (RATIO - 1.0) / (RATIO ** (NUM_ELMT / 2) - 1.0)
    idx = np.arange(-NUM_ELMT // 2, NUM_ELMT // 2 + 1).astype(np.float64)
    sign = np.sign(idx)
    mesh = (RATIO ** np.abs(idx) - 1.0) / (RATIO - 1.0) * one_step * sign
    mesh_norm = (mesh + BOUND) / 2.0 / BOUND
    mesh_norm = np.concatenate([[0.0], mesh_norm[1:-1], [1.0]])
    elmt_size = mesh_norm[1:] - mesh_norm[:-1]
    return mesh_norm.astype(np.float32), elmt_size.astype(np.float32)


_MESH_NORM, _ELMT_SIZE = _mesh_tables()
_CF = (_ELMT_SIZE[:-1] + _ELMT_SIZE[1:]) / np.float32(2.0)
_C1 = np.float32(1.0) - _ELMT_SIZE[0]
_XS64 = (np.float32(100.0) * _MESH_NORM[:NUM_ELMT] - np.float32(50.0))
_H64 = np.float32(2.0) * _ELMT_SIZE


def _body(x_hbm, logp_hbm, xs_hbm, h_hbm, out_hbm,
          logp_v, yr_tab, pdf_tab, a_tab, b_tab, xs_tab, h_tab, xbuf, obuf):
    wid = lax.axis_index("s") * NC + lax.axis_index("c")
    c0 = wid * COLS
    lanes = lax.iota(jnp.int32, L)
    lane_lo = lanes + L
    lane_hi = lanes + NUM_ELMT * L

    pltpu.sync_copy(logp_hbm.at[:, pl.ds(c0, COLS)], logp_v)
    pltpu.sync_copy(xs_hbm, xs_tab)
    pltpu.sync_copy(h_hbm, h_tab)

    S = jnp.zeros((L,), jnp.float32)
    for m in range(NUM_ELMT - 1):
        wv = jnp.exp(logp_v[m, :])
        pdf_tab[pl.ds((m + 1) * L, L)] = wv
        S = S + wv * float(_CF[m])
    inv = float(_C1) / S
    one_v = jnp.ones((L,), jnp.float32)
    pdf_tab[pl.ds(0, L)] = one_v
    pdf_tab[pl.ds(NUM_ELMT * L, L)] = one_v
    for m in range(NUM_ELMT - 1):
        pdf_tab[pl.ds((m + 1) * L, L)] = pdf_tab[pl.ds((m + 1) * L, L)] * inv
    yr_tab[pl.ds(0, L)] = jnp.full((L,), -50.0, jnp.float32)
    F = jnp.zeros((L,), jnp.float32)
    prev = pdf_tab[pl.ds(0, L)]
    for s in range(NUM_ELMT):
        cur = pdf_tab[pl.ds((s + 1) * L, L)]
        a_tab[pl.ds(s * L, L)] = prev * float(_ELMT_SIZE[s])
        b_tab[pl.ds(s * L, L)] = (cur - prev) * float(0.02 * _ELMT_SIZE[s])
        if s < NUM_ELMT - 1:
            F = F + (prev + cur) * float(0.5 * _ELMT_SIZE[s])
            yr_tab[pl.ds((s + 1) * L, L)] = F * 100.0 - 50.0
        prev = cur
    yr_tab[pl.ds(NUM_ELMT * L, L)] = jnp.full((L,), 50.0, jnp.float32)
    inf_v = jnp.full((L,), jnp.inf, jnp.float32)
    for m in range(NUM_ELMT + 1, YR_ROWS):
        yr_tab[pl.ds(m * L, L)] = inf_v

    def chunk_body(t, carry):
        r0 = t * R
        pltpu.sync_copy(x_hbm.at[pl.ds(r0, R), pl.ds(c0, COLS)], xbuf)

        @plsc.parallel_loop(0, R, step=1, unroll=4)
        def row_body(i):
            xv = xbuf[i, :]
            K = lanes
            for b in (64, 32, 16, 8, 4, 2, 1):
                val = plsc.load_gather(yr_tab, [K + ((b - 1) * L)])
                K = jnp.where(val < xv, K + b * L, K)
            cover = (K >= L) & (K < (NUM_ELMT + 1) * L)
            Km1 = jnp.minimum(jnp.maximum(K, lane_lo), lane_hi) - L
            ykx = plsc.load_gather(yr_tab, [Km1])
            A = plsc.load_gather(a_tab, [Km1])
            B = plsc.load_gather(b_tab, [Km1])
            H = plsc.load_gather(h_tab, [Km1])
            XS = plsc.load_gather(xs_tab, [Km1])
            xr = xv - ykx
            dm = jnp.maximum(B * xr + A * A, 1e-30)
            iv = 0x5F3759DF - lax.shift_right_logical(
                lax.bitcast_convert_type(dm, jnp.int32), 1)
            rs = lax.bitcast_convert_type(iv, jnp.float32)
            hd = 0.5 * dm
            rs = rs * (1.5 - hd * rs * rs)
            rs = rs * (1.5 - hd * rs * rs)
            tp = (H * xr) / (dm * rs + A) + XS
            obuf[i, :] = jnp.where(cover, tp, xv)

        pltpu.sync_copy(obuf, out_hbm.at[pl.ds(r0, R), pl.ds(c0, COLS)])
        return carry

    lax.fori_loop(0, NCHUNK, chunk_body, 0)


_sc_call = pl.kernel(
    _body,
    out_type=jax.ShapeDtypeStruct((N, INPUT_DIM), jnp.float32),
    mesh=plsc.VectorSubcoreMesh(core_axis_name="c", subcore_axis_name="s"),
    compiler_params=pltpu.CompilerParams(
        use_tc_tiling_on_sc=False, needs_layout_passes=False),
    scratch_types=[
        pltpu.VMEM((NUM_ELMT - 1, COLS), jnp.float32),
        pltpu.VMEM((YR_ROWS * L,), jnp.float32),
        pltpu.VMEM(((NUM_ELMT + 1) * L,), jnp.float32),
        pltpu.VMEM((NUM_ELMT * L,), jnp.float32),
        pltpu.VMEM((NUM_ELMT * L,), jnp.float32),
        pltpu.VMEM((NUM_ELMT * L,), jnp.float32),
        pltpu.VMEM((NUM_ELMT * L,), jnp.float32),
        pltpu.VMEM((R, COLS), jnp.float32),
        pltpu.VMEM((R, COLS), jnp.float32),
    ],
)


def kernel(x, log_p):
    xs_c = jnp.asarray(np.tile(_XS64[:, None], (1, L)).reshape(-1))
    h_c = jnp.asarray(np.tile(_H64[:, None], (1, L)).reshape(-1))
    return _sc_call(x, log_p, xs_c, h_c)
